# baseline (speedup 1.0000x reference)
import functools

import jax
import jax.numpy as jnp
from jax import lax
from jax.experimental import pallas as pl
from jax.experimental.pallas import tpu as pltpu
from jax.experimental.pallas import tpu_sc as plsc

E = 128
S = 200
B = 4096
SP = 208
HALF = 104
L = 16
NC = 2
NS = 16
NW = NC * NS
BPW = B // NW
NFULL = S // L
TAIL = S - NFULL * L


def _tile_body(tok_hbm, table_hbm, out_hbm, idx_v, rows_v, out_v, gsem):
    wid = lax.axis_index("s") * NC + lax.axis_index("c")
    base_b = wid * BPW
    pltpu.sync_copy(tok_hbm.at[pl.ds(base_b, BPW)], idx_v)

    lane = lax.iota(jnp.int32, L)
    tail_mask = lane < TAIL

    @pl.loop(0, BPW)
    def _b_loop(b):
        g0 = pltpu.async_copy(
            table_hbm.at[idx_v.at[b, pl.ds(0, HALF)]],
            rows_v.at[pl.ds(0, HALF)], gsem)
        g1 = pltpu.async_copy(
            table_hbm.at[idx_v.at[b, pl.ds(HALF, HALF)]],
            rows_v.at[pl.ds(HALF, HALF)], gsem)
        g0.wait()
        g1.wait()

        @pl.loop(0, E)
        def _e_loop(e):
            col = jnp.broadcast_to(e, (L,))
            for k in range(NFULL):
                v = plsc.load_gather(rows_v, [lane + k * L, col])
                out_v[e, pl.ds(k * L, L)] = v
            v = plsc.load_gather(rows_v, [lane + NFULL * L, col])
            plsc.store_scatter(out_v, [col, lane + NFULL * L], v,
                               mask=tail_mask)

        pltpu.sync_copy(out_v, out_hbm.at[base_b + b])


@functools.partial(
    pl.kernel,
    out_type=jax.ShapeDtypeStruct((B, E, S), jnp.float32),
    mesh=plsc.VectorSubcoreMesh(core_axis_name="c", subcore_axis_name="s"),
    compiler_params=pltpu.CompilerParams(use_tc_tiling_on_sc=False,
                                         needs_layout_passes=False),
    scratch_types=[
        pltpu.VMEM((BPW, SP), jnp.int32),
        pltpu.VMEM((SP, E), jnp.float32),
        pltpu.VMEM((E, S), jnp.float32),
        pltpu.SemaphoreType.DMA,
    ],
)
def _bow_encode(tok_hbm, table_hbm, out_hbm, idx_v, rows_v, out_v, gsem):
    _tile_body(tok_hbm, table_hbm, out_hbm, idx_v, rows_v, out_v, gsem)


def kernel(tokens, table):
    tokens_p = jnp.pad(tokens.astype(jnp.int32).T, ((0, 0), (0, SP - S)))
    return _bow_encode(tokens_p, table)


# speedup vs baseline: 1.1076x; 1.1076x over previous
import functools

import jax
import jax.numpy as jnp
from jax import lax
from jax.experimental import pallas as pl
from jax.experimental.pallas import tpu as pltpu
from jax.experimental.pallas import tpu_sc as plsc

E = 128
S = 200
B = 4096
C0 = 104
C1 = S - C0
L = 16
NC = 2
NS = 16
NW = NC * NS
BPW = B // NW
NFULL = S // L
TAIL = S - NFULL * L


def _tile_body(tok_hbm, table_hbm, out_hbm, idx_v, rows0, rows1, out0, out1,
               gsem0, gsem1, osem0, osem1):
    wid = lax.axis_index("s") * NC + lax.axis_index("c")
    base_b = wid * BPW
    pltpu.sync_copy(tok_hbm.at[pl.ds(base_b, BPW)], idx_v)

    lane = lax.iota(jnp.int32, L)
    tail_mask = lane < TAIL
    tail_rows = jnp.minimum(lane + NFULL * L, S - 1)

    def gather(b, rows_v, sem):
        return (
            pltpu.make_async_copy(
                table_hbm.at[idx_v.at[b, pl.ds(0, C0)]],
                rows_v.at[pl.ds(0, C0)], sem),
            pltpu.make_async_copy(
                table_hbm.at[idx_v.at[b, pl.ds(C0, C1)]],
                rows_v.at[pl.ds(C0, C1)], sem),
        )

    def outcopy(b, out_v, sem):
        return pltpu.make_async_copy(out_v, out_hbm.at[base_b + b], sem)

    def transpose(rows_v, out_v):
        @pl.loop(0, E)
        def _e_loop(e):
            col = jnp.broadcast_to(e, (L,))
            for k in range(NFULL):
                v = plsc.load_gather(rows_v, [lane + k * L, col])
                out_v[e, pl.ds(k * L, L)] = v
            v = plsc.load_gather(rows_v, [tail_rows, col])
            plsc.store_scatter(out_v, [col, lane + NFULL * L], v,
                               mask=tail_mask)

    bufs = ((rows0, out0, gsem0, osem0), (rows1, out1, gsem1, osem1))

    for c in gather(0, rows0, gsem0):
        c.start()

    @pl.loop(0, BPW, step=2)
    def _b_loop(b):
        for h in range(2):
            rows_v, out_v, gsem, osem = bufs[h]
            bb = b + h

            @pl.when(bb + 1 < BPW)
            def _():
                nxt = bufs[(h + 1) % 2]
                for c in gather(bb + 1, nxt[0], nxt[2]):
                    c.start()

            for c in gather(bb, rows_v, gsem):
                c.wait()

            @pl.when(bb >= 2)
            def _():
                outcopy(bb - 2, out_v, osem).wait()

            transpose(rows_v, out_v)
            outcopy(bb, out_v, osem).start()

    outcopy(BPW - 2, out0, osem0).wait()
    outcopy(BPW - 1, out1, osem1).wait()


@functools.partial(
    pl.kernel,
    out_type=jax.ShapeDtypeStruct((B, E, S), jnp.float32),
    mesh=plsc.VectorSubcoreMesh(core_axis_name="c", subcore_axis_name="s"),
    compiler_params=pltpu.CompilerParams(use_tc_tiling_on_sc=False,
                                         needs_layout_passes=False),
    scratch_types=[
        pltpu.VMEM((BPW, S), jnp.int32),
        pltpu.VMEM((S, E), jnp.float32),
        pltpu.VMEM((S, E), jnp.float32),
        pltpu.VMEM((E, S), jnp.float32),
        pltpu.VMEM((E, S), jnp.float32),
        pltpu.SemaphoreType.DMA,
        pltpu.SemaphoreType.DMA,
        pltpu.SemaphoreType.DMA,
        pltpu.SemaphoreType.DMA,
    ],
)
def _bow_encode(tok_hbm, table_hbm, out_hbm, idx_v, rows0, rows1, out0, out1,
                gsem0, gsem1, osem0, osem1):
    _tile_body(tok_hbm, table_hbm, out_hbm, idx_v, rows0, rows1, out0, out1,
               gsem0, gsem1, osem0, osem1)


def kernel(tokens, table):
    return _bow_encode(tokens.astype(jnp.int32).T, table)


# speedup vs baseline: 1.6366x; 1.4776x over previous
import functools

import jax
import jax.numpy as jnp
from jax import lax
from jax.experimental import pallas as pl
from jax.experimental.pallas import tpu as pltpu
from jax.experimental.pallas import tpu_sc as plsc

E = 128
S = 200
B = 4096
C0 = 104
C1 = S - C0
L = 16
NC = 2
NS = 16
NW = NC * NS
BPW = B // NW
HB = BPW // 2
NFULL = S // L
TAIL = S - NFULL * L


def _tile_body(tok_hbm, table_hbm, out_hbm, tokt_v, idx_v, rows0, rows1,
               out0, out1, gsem0, gsem1, osem0, osem1):
    wid = lax.axis_index("s") * NC + lax.axis_index("c")

    lane = lax.iota(jnp.int32, L)
    tail_mask = lane < TAIL
    tail_rows = jnp.minimum(lane + NFULL * L, S - 1)

    def transpose(src_v, dst_v, nrows):
        @plsc.parallel_loop(0, nrows, unroll=2)
        def _r_loop(r):
            col = jnp.broadcast_to(r, (L,))
            for k in range(NFULL):
                v = plsc.load_gather(src_v, [lane + k * L, col])
                dst_v[r, pl.ds(k * L, L)] = v
            v = plsc.load_gather(src_v, [tail_rows, col])
            plsc.store_scatter(dst_v, [col, lane + NFULL * L], v,
                               mask=tail_mask)

    def gather(b, rows_v, sem):
        return (
            pltpu.make_async_copy(
                table_hbm.at[idx_v.at[b, pl.ds(0, C0)]],
                rows_v.at[pl.ds(0, C0)], sem),
            pltpu.make_async_copy(
                table_hbm.at[idx_v.at[b, pl.ds(C0, C1)]],
                rows_v.at[pl.ds(C0, C1)], sem),
        )

    for p in range(2):
        base = wid * BPW + p * HB

        pltpu.sync_copy(tok_hbm.at[:, pl.ds(base, HB)], tokt_v)
        transpose(tokt_v, idx_v, HB)

        def outcopy(b, out_v, sem):
            return pltpu.make_async_copy(out_v, out_hbm.at[base + b], sem)

        bufs = ((rows0, out0, gsem0, osem0), (rows1, out1, gsem1, osem1))

        for c in gather(0, rows0, gsem0):
            c.start()

        @pl.loop(0, HB, step=2)
        def _b_loop(b):
            for h in range(2):
                rows_v, out_v, gsem, osem = bufs[h]
                bb = b + h

                @pl.when(bb + 1 < HB)
                def _():
                    nxt = bufs[(h + 1) % 2]
                    for c in gather(bb + 1, nxt[0], nxt[2]):
                        c.start()

                for c in gather(bb, rows_v, gsem):
                    c.wait()

                @pl.when(bb >= 2)
                def _():
                    outcopy(bb - 2, out_v, osem).wait()

                transpose(rows_v, out_v, E)
                outcopy(bb, out_v, osem).start()

        outcopy(HB - 2, out0, osem0).wait()
        outcopy(HB - 1, out1, osem1).wait()


@functools.partial(
    pl.kernel,
    out_type=jax.ShapeDtypeStruct((B, E, S), jnp.float32),
    mesh=plsc.VectorSubcoreMesh(core_axis_name="c", subcore_axis_name="s"),
    compiler_params=pltpu.CompilerParams(use_tc_tiling_on_sc=False,
                                         needs_layout_passes=False),
    scratch_types=[
        pltpu.VMEM((S, HB), jnp.int32),
        pltpu.VMEM((HB, S), jnp.int32),
        pltpu.VMEM((S, E), jnp.float32),
        pltpu.VMEM((S, E), jnp.float32),
        pltpu.VMEM((E, S), jnp.float32),
        pltpu.VMEM((E, S), jnp.float32),
        pltpu.SemaphoreType.DMA,
        pltpu.SemaphoreType.DMA,
        pltpu.SemaphoreType.DMA,
        pltpu.SemaphoreType.DMA,
    ],
)
def _bow_encode(tok_hbm, table_hbm, out_hbm, tokt_v, idx_v, rows0, rows1,
                out0, out1, gsem0, gsem1, osem0, osem1):
    _tile_body(tok_hbm, table_hbm, out_hbm, tokt_v, idx_v, rows0, rows1,
               out0, out1, gsem0, gsem1, osem0, osem1)


def kernel(tokens, table):
    return _bow_encode(tokens.astype(jnp.int32), table)


# speedup vs baseline: 12.6030x; 7.7007x over previous
import functools

import jax
import jax.numpy as jnp
from jax import lax
from jax.experimental import pallas as pl
from jax.experimental.pallas import tpu as pltpu
from jax.experimental.pallas import tpu_sc as plsc

E = 128
S = 200
B = 4096
C0 = 104
C1 = S - C0
L = 16
NC = 2
NS = 16
NW = NC * NS
BPW = B // NW
NFULL = S // L
TAIL = S - NFULL * L
NBUF = 3


def _tile_body(tok_hbm, table_hbm, out_hbm, tokt_v, idx_v, rows0, rows1,
               rows2, gsem0, gsem1, gsem2, osem0, osem1, osem2):
    wid = lax.axis_index("s") * NC + lax.axis_index("c")
    base = wid * BPW

    lane = lax.iota(jnp.int32, L)
    tail_mask = lane < TAIL
    tail_rows = jnp.minimum(lane + NFULL * L, S - 1)

    pltpu.sync_copy(tok_hbm.at[:, pl.ds(base, BPW)], tokt_v)

    @plsc.parallel_loop(0, BPW, unroll=2)
    def _r_loop(r):
        col = jnp.broadcast_to(r, (L,))
        for k in range(NFULL):
            v = plsc.load_gather(tokt_v, [lane + k * L, col])
            idx_v[r, pl.ds(k * L, L)] = v
        v = plsc.load_gather(tokt_v, [tail_rows, col])
        plsc.store_scatter(idx_v, [col, lane + NFULL * L], v, mask=tail_mask)

    bufs = (rows0, rows1, rows2)
    gsems = (gsem0, gsem1, gsem2)
    osems = (osem0, osem1, osem2)

    def gather(b, j):
        return (
            pltpu.make_async_copy(
                table_hbm.at[idx_v.at[b, pl.ds(0, C0)]],
                bufs[j].at[pl.ds(0, C0)], gsems[j]),
            pltpu.make_async_copy(
                table_hbm.at[idx_v.at[b, pl.ds(C0, C1)]],
                bufs[j].at[pl.ds(C0, C1)], gsems[j]),
        )

    def outcopy(b, j):
        return pltpu.make_async_copy(bufs[j], out_hbm.at[base + b], osems[j])

    def slot(bb, h):
        for c in gather(bb, h):
            c.wait()
        outcopy(bb, h).start()
        j2 = (h + 2) % NBUF

        @pl.when(bb + 2 < BPW)
        def _():
            @pl.when(bb >= 1)
            def _():
                outcopy(bb - 1, j2).wait()

            for c in gather(bb + 2, j2):
                c.start()

    for c in gather(0, 0) + gather(1, 1):
        c.start()

    @pl.loop(0, BPW - 2, step=NBUF)
    def _b_loop(b):
        for h in range(NBUF):
            slot(b + h, h)

    slot(BPW - 2, 0)
    slot(BPW - 1, 1)

    outcopy(BPW - 3, 2).wait()
    outcopy(BPW - 2, 0).wait()
    outcopy(BPW - 1, 1).wait()


@functools.partial(
    pl.kernel,
    out_type=jax.ShapeDtypeStruct((B, S, E), jnp.float32),
    mesh=plsc.VectorSubcoreMesh(core_axis_name="c", subcore_axis_name="s"),
    compiler_params=pltpu.CompilerParams(use_tc_tiling_on_sc=False,
                                         needs_layout_passes=False),
    scratch_types=[
        pltpu.VMEM((S, BPW), jnp.int32),
        pltpu.VMEM((BPW, S), jnp.int32),
        pltpu.VMEM((S, E), jnp.float32),
        pltpu.VMEM((S, E), jnp.float32),
        pltpu.VMEM((S, E), jnp.float32),
        pltpu.SemaphoreType.DMA,
        pltpu.SemaphoreType.DMA,
        pltpu.SemaphoreType.DMA,
        pltpu.SemaphoreType.DMA,
        pltpu.SemaphoreType.DMA,
        pltpu.SemaphoreType.DMA,
    ],
)
def _bow_encode(tok_hbm, table_hbm, out_hbm, tokt_v, idx_v, rows0, rows1,
                rows2, gsem0, gsem1, gsem2, osem0, osem1, osem2):
    _tile_body(tok_hbm, table_hbm, out_hbm, tokt_v, idx_v, rows0, rows1,
               rows2, gsem0, gsem1, gsem2, osem0, osem1, osem2)


def kernel(tokens, table):
    out = _bow_encode(tokens.astype(jnp.int32), table)
    return jnp.swapaxes(out, 1, 2)
